# Initial kernel scaffold; baseline (speedup 1.0000x reference)
import functools

import jax
import jax.numpy as jnp
from jax import lax
from jax.experimental import pallas as pl
from jax.experimental.pallas import tpu as pltpu
from jax.experimental.pallas import tpu_sc as plsc

NUM_CORES = 2
NUM_SUBCORES = 16
NUM_WORKERS = NUM_CORES * NUM_SUBCORES
CHUNK = 2048


@functools.partial(jax.jit, static_argnames=("B", "D"))
def _gather_flat(idx_flat, W, B, D):
    per_w = B // NUM_WORKERS
    n_chunks = per_w // CHUNK
    mesh = plsc.VectorSubcoreMesh(
        core_axis_name="c",
        subcore_axis_name="s",
        num_cores=NUM_CORES,
        num_subcores=NUM_SUBCORES,
    )

    @functools.partial(
        pl.kernel,
        mesh=mesh,
        out_type=jax.ShapeDtypeStruct((B, D), jnp.float32),
        scratch_types=[
            pltpu.VMEM((2, CHUNK), jnp.int32),
            pltpu.VMEM((2, CHUNK, D), jnp.float32),
            pltpu.SemaphoreType.DMA,
        ],
    )
    def k(idx_hbm, table_hbm, out_hbm, idx_v, rows_v, sem):
        wid = lax.axis_index("s") * NUM_CORES + lax.axis_index("c")
        base = wid * per_w

        def body(g, carry):
            off = base + g * CHUNK
            pltpu.sync_copy(idx_hbm.at[pl.ds(off, CHUNK)], idx_v.at[0])
            pltpu.async_copy(table_hbm.at[idx_v.at[0]], rows_v.at[0], sem).wait()
            pltpu.sync_copy(rows_v.at[0], out_hbm.at[pl.ds(off, CHUNK)])
            return carry

        lax.fori_loop(0, n_chunks, body, 0)

    return k(idx_flat, W)


def kernel(x, W):
    Bx, H = x.shape
    V, D = W.shape
    B = Bx * H
    out = _gather_flat(x.reshape(B), W, B, D)
    return out.reshape(Bx, H, D)


# speedup vs baseline: 2.4904x; 2.4904x over previous
import functools

import jax
import jax.numpy as jnp
from jax import lax
from jax.experimental import pallas as pl
from jax.experimental.pallas import tpu as pltpu
from jax.experimental.pallas import tpu_sc as plsc

NUM_CORES = 2
NUM_SUBCORES = 16
NUM_WORKERS = NUM_CORES * NUM_SUBCORES
CHUNK = 2048


@functools.partial(jax.jit, static_argnames=("B", "D"))
def _gather_flat(idx_flat, W, B, D):
    per_w = B // NUM_WORKERS
    n_chunks = per_w // CHUNK
    mesh = plsc.VectorSubcoreMesh(
        core_axis_name="c",
        subcore_axis_name="s",
        num_cores=NUM_CORES,
        num_subcores=NUM_SUBCORES,
    )

    @functools.partial(
        pl.kernel,
        mesh=mesh,
        compiler_params=pltpu.CompilerParams(use_tc_tiling_on_sc=False),
        out_type=jax.ShapeDtypeStruct((B, D), jnp.float32),
        scratch_types=[
            pltpu.VMEM((2, CHUNK), jnp.int32),
            pltpu.VMEM((2, CHUNK, D), jnp.float32),
            pltpu.SemaphoreType.DMA,
        ],
    )
    def k(idx_hbm, table_hbm, out_hbm, idx_v, rows_v, sem):
        wid = lax.axis_index("s") * NUM_CORES + lax.axis_index("c")
        base = wid * per_w

        def body(g, carry):
            off = base + g * CHUNK
            pltpu.sync_copy(idx_hbm.at[pl.ds(off, CHUNK)], idx_v.at[0])
            pltpu.async_copy(table_hbm.at[idx_v.at[0]], rows_v.at[0], sem).wait()
            pltpu.sync_copy(rows_v.at[0], out_hbm.at[pl.ds(off, CHUNK)])
            return carry

        lax.fori_loop(0, n_chunks, body, 0)

    return k(idx_flat, W)


def kernel(x, W):
    Bx, H = x.shape
    V, D = W.shape
    B = Bx * H
    out = _gather_flat(x.reshape(B), W, B, D)
    return out.reshape(Bx, H, D)


# speedup vs baseline: 2.5329x; 1.0171x over previous
import functools

import jax
import jax.numpy as jnp
from jax import lax
from jax.experimental import pallas as pl
from jax.experimental.pallas import tpu as pltpu
from jax.experimental.pallas import tpu_sc as plsc

NUM_CORES = 2
NUM_SUBCORES = 16
NUM_WORKERS = NUM_CORES * NUM_SUBCORES
CHUNK = 2048


@functools.partial(jax.jit, static_argnames=("B", "D"))
def _gather_flat(idx_flat, W, B, D):
    per_w = B // NUM_WORKERS
    n_chunks = per_w // CHUNK
    assert n_chunks % 2 == 0 and n_chunks >= 4
    n_pairs = (n_chunks - 2) // 2
    mesh = plsc.VectorSubcoreMesh(
        core_axis_name="c",
        subcore_axis_name="s",
        num_cores=NUM_CORES,
        num_subcores=NUM_SUBCORES,
    )

    @functools.partial(
        pl.kernel,
        mesh=mesh,
        compiler_params=pltpu.CompilerParams(use_tc_tiling_on_sc=False),
        out_type=jax.ShapeDtypeStruct((B, D), jnp.float32),
        scratch_types=[
            pltpu.VMEM((2, CHUNK), jnp.int32),
            pltpu.VMEM((2, CHUNK, D), jnp.float32),
            pltpu.SemaphoreType.DMA,
            pltpu.SemaphoreType.DMA,
        ],
    )
    def k(idx_hbm, table_hbm, out_hbm, idx_v, rows_v, gsem, osem):
        wid = lax.axis_index("s") * NUM_CORES + lax.axis_index("c")
        base = wid * per_w

        def load_idx(g, b):
            pltpu.sync_copy(idx_hbm.at[pl.ds(base + g * CHUNK, CHUNK)],
                            idx_v.at[b])

        def fire_gather(b):
            pltpu.async_copy(table_hbm.at[idx_v.at[b]], rows_v.at[b], gsem)

        def wait_gather(b):
            pltpu.make_async_copy(table_hbm.at[idx_v.at[b]], rows_v.at[b],
                                  gsem).wait()

        def fire_store(g, b):
            pltpu.async_copy(rows_v.at[b],
                             out_hbm.at[pl.ds(base + g * CHUNK, CHUNK)], osem)

        def wait_store(g, b):
            pltpu.make_async_copy(rows_v.at[b],
                                  out_hbm.at[pl.ds(base + g * CHUNK, CHUNK)],
                                  osem).wait()

        load_idx(0, 0)
        fire_gather(0)
        load_idx(1, 1)
        wait_gather(0)
        fire_gather(1)
        fire_store(0, 0)

        def step(g, b):
            load_idx(g + 1, b ^ 1)
            wait_gather(b)
            wait_store(g - 1, b ^ 1)
            fire_gather(b ^ 1)
            fire_store(g, b)

        def body(p, carry):
            step(2 * p + 1, 1)
            step(2 * p + 2, 0)
            return carry

        lax.fori_loop(0, n_pairs, body, 0)

        g_last = n_chunks - 1
        wait_gather(1)
        wait_store(g_last - 1, 0)
        fire_store(g_last, 1)
        wait_store(g_last, 1)

    return k(idx_flat, W)


def kernel(x, W):
    Bx, H = x.shape
    V, D = W.shape
    B = Bx * H
    out = _gather_flat(x.reshape(B), W, B, D)
    return out.reshape(Bx, H, D)


# speedup vs baseline: 4.1154x; 1.6248x over previous
import functools

import jax
import jax.numpy as jnp
from jax import lax
from jax.experimental import pallas as pl
from jax.experimental.pallas import tpu as pltpu
from jax.experimental.pallas import tpu_sc as plsc

NUM_CORES = 2
NUM_SUBCORES = 16
NUM_WORKERS = NUM_CORES * NUM_SUBCORES
CHUNK = 2048


def _mesh():
    return plsc.VectorSubcoreMesh(
        core_axis_name="c",
        subcore_axis_name="s",
        num_cores=NUM_CORES,
        num_subcores=NUM_SUBCORES,
    )


@functools.partial(jax.jit, static_argnames=("B", "D"))
def _gather_flat(idx_flat, W, B, D):
    per_w = B // NUM_WORKERS
    n_chunks = per_w // CHUNK
    assert n_chunks % 2 == 0 and n_chunks >= 4
    n_pairs = (n_chunks - 2) // 2

    @functools.partial(
        pl.kernel,
        mesh=_mesh(),
        compiler_params=pltpu.CompilerParams(use_tc_tiling_on_sc=False),
        out_type=jax.ShapeDtypeStruct((B, D), jnp.float32),
        scratch_types=[
            pltpu.VMEM((2, CHUNK), jnp.int32),
            pltpu.VMEM((2, CHUNK, D), jnp.float32),
            pltpu.SemaphoreType.DMA,
            pltpu.SemaphoreType.DMA,
        ],
    )
    def k(idx_hbm, table_hbm, out_hbm, idx_v, rows_v, gsem, osem):
        wid = lax.axis_index("s") * NUM_CORES + lax.axis_index("c")
        base = wid * per_w

        def load_idx(g, b):
            pltpu.sync_copy(idx_hbm.at[pl.ds(base + g * CHUNK, CHUNK)],
                            idx_v.at[b])

        def fire_gather(b):
            pltpu.async_copy(table_hbm.at[idx_v.at[b]], rows_v.at[b], gsem)

        def wait_gather(b):
            pltpu.make_async_copy(table_hbm.at[idx_v.at[b]], rows_v.at[b],
                                  gsem).wait()

        def fire_store(g, b):
            pltpu.async_copy(rows_v.at[b],
                             out_hbm.at[pl.ds(base + g * CHUNK, CHUNK)], osem)

        def wait_store(g, b):
            pltpu.make_async_copy(rows_v.at[b],
                                  out_hbm.at[pl.ds(base + g * CHUNK, CHUNK)],
                                  osem).wait()

        load_idx(0, 0)
        fire_gather(0)
        load_idx(1, 1)
        wait_gather(0)
        fire_gather(1)
        fire_store(0, 0)

        def step(g, b):
            load_idx(g + 1, b ^ 1)
            wait_gather(b)
            wait_store(g - 1, b ^ 1)
            fire_gather(b ^ 1)
            fire_store(g, b)

        def body(p, carry):
            step(2 * p + 1, 1)
            step(2 * p + 2, 0)
            return carry

        lax.fori_loop(0, n_pairs, body, 0)

        g_last = n_chunks - 1
        wait_gather(1)
        wait_store(g_last - 1, 0)
        fire_store(g_last, 1)
        wait_store(g_last, 1)

    return k(idx_flat, W)


UNITS = 1600
UNIT_ROWS = 2048
UNIT_ELEMS = UNIT_ROWS * 16


@functools.partial(jax.jit, static_argnames=("HIST", "BATCH"))
def _relayout(flat, HIST, BATCH):
    per_w = UNITS // NUM_WORKERS

    @functools.partial(
        pl.kernel,
        mesh=_mesh(),
        compiler_params=pltpu.CompilerParams(
            use_tc_tiling_on_sc=True, needs_layout_passes=False),
        out_type=jax.ShapeDtypeStruct((HIST, 16, BATCH), jnp.float32),
        scratch_types=[
            pltpu.VMEM((2, UNIT_ELEMS), jnp.float32),
            pltpu.VMEM((8, 2, 1, 8, 256), jnp.float32),
            pltpu.SemaphoreType.DMA,
            pltpu.SemaphoreType.DMA,
        ],
    )
    def k(in_hbm, out_hbm, buf, tbuf, lsem, osem):
        wid = lax.axis_index("s") * NUM_CORES + lax.axis_index("c")
        ubase = wid * per_w
        lane16 = lax.iota(jnp.int32, 16) * 16

        def fire_load(u, b):
            pltpu.async_copy(
                in_hbm.at[pl.ds((ubase + u) * UNIT_ELEMS, UNIT_ELEMS)],
                buf.at[b], lsem)

        def wait_load(u, b):
            pltpu.make_async_copy(
                in_hbm.at[pl.ds((ubase + u) * UNIT_ELEMS, UNIT_ELEMS)],
                buf.at[b], lsem).wait()

        def transpose(b):
            b_vec = jnp.full((16,), b, jnp.int32)

            def p_body(p, carry):
                jr = p // 16
                icl = (p % 16) // 8
                ii16 = p % 8
                row0 = (icl * 8 + jr) * 128 + ii16 * 16
                col = icl * 128 + ii16 * 16
                for tr in range(2):
                    for dd in range(8):
                        idx = lane16 + (row0 * 16 + 8 * tr + dd)
                        v = plsc.load_gather(buf, [b_vec, idx])
                        tbuf[jr, tr, 0, dd, pl.ds(col, 16)] = v
                return carry
            lax.fori_loop(0, 128, p_body, 0)

        def out_slice(u, jr, tr):
            ug = ubase + u
            jt = ug // 64
            ic0 = 2 * (ug % 64)
            j = jt * 8 + jr
            return out_hbm.at[pl.ds(j, 1), pl.ds(8 * tr, 8),
                              pl.ds(ic0 * 128, 256)]

        def fire_stores(u):
            for jr in range(8):
                for tr in range(2):
                    pltpu.async_copy(tbuf.at[jr, tr], out_slice(u, jr, tr),
                                     osem)

        def wait_stores(u):
            for jr in range(8):
                for tr in range(2):
                    pltpu.make_async_copy(tbuf.at[jr, tr],
                                          out_slice(u, jr, tr), osem).wait()

        fire_load(0, 0)
        fire_load(1, 1)
        wait_load(0, 0)
        transpose(0)
        fire_stores(0)

        def body(u, carry):
            b = u % 2
            fire_load(u + 1, b ^ 1)
            wait_load(u, b)
            wait_stores(u - 1)
            transpose(b)
            fire_stores(u)
            return carry

        lax.fori_loop(1, per_w - 1, body, 0)

        u_last = per_w - 1
        wait_load(u_last, u_last % 2)
        wait_stores(u_last - 1)
        transpose(u_last % 2)
        fire_stores(u_last)
        wait_stores(u_last)

    return k(flat)


def kernel(x, W):
    Bx, H = x.shape
    V, D = W.shape
    B = Bx * H
    idx_flat = (x.reshape(Bx // 128, 128, H // 8, 8)
                .transpose(2, 0, 3, 1).reshape(B))
    out_lin = _gather_flat(idx_flat, W, B, D)
    z = _relayout(out_lin.reshape(B * D), H, Bx)
    return jnp.transpose(z, (2, 0, 1))
